# baseline (speedup 1.0000x reference)
import jax
import jax.numpy as jnp
from jax import lax
from jax.experimental import pallas as pl
from jax.experimental.pallas import tpu as pltpu
from jax.experimental.pallas import tpu_sc as plsc

_NC = 2
_NS = 16
_NW = _NC * _NS
_CHUNK = 128
_L = 16


def _sc_gather_body(uidx_hbm, pidx_hbm, uridx_hbm, pridx_hbm,
                    uemb_hbm, pemb_hbm, ubias16_hbm, pbias16_hbm,
                    u_out, p_out, ub_out, pb_out,
                    uidx_v, pidx_v, uridx_v, pridx_v,
                    urows_v, prows_v, ubrows_v, pbrows_v,
                    ubvals_v, pbvals_v, sem_e, sem_b):
    k = uidx_v.shape[0]
    bpw = k * _CHUNK
    wid = lax.axis_index("s") * _NC + lax.axis_index("c")
    base = wid * bpw
    pltpu.sync_copy(uidx_hbm.at[wid], uidx_v)
    pltpu.sync_copy(pidx_hbm.at[wid], pidx_v)
    pltpu.sync_copy(uridx_hbm.at[wid], uridx_v)
    pltpu.sync_copy(pridx_hbm.at[wid], pridx_v)
    emb_copies = []
    bias_copies = []
    for j in range(k):
        sl = pl.ds(j * _CHUNK, _CHUNK)
        emb_copies.append(pltpu.async_copy(
            uemb_hbm.at[uidx_v.at[j]], urows_v.at[sl], sem_e))
        emb_copies.append(pltpu.async_copy(
            pemb_hbm.at[pidx_v.at[j]], prows_v.at[sl], sem_e))
        bias_copies.append(pltpu.async_copy(
            ubias16_hbm.at[uridx_v.at[j]], ubrows_v.at[sl], sem_b))
        bias_copies.append(pltpu.async_copy(
            pbias16_hbm.at[pridx_v.at[j]], pbrows_v.at[sl], sem_b))
    for c in bias_copies:
        c.wait()
    lane_iota = lax.iota(jnp.int32, _L)
    for j in range(k):
        for c in range(_CHUNK // _L):
            off = j * _CHUNK + c * _L
            jvec = off + lane_iota
            usl = uidx_v.at[j][pl.ds(c * _L, _L)] & (_L - 1)
            psl = pidx_v.at[j][pl.ds(c * _L, _L)] & (_L - 1)
            ubvals_v[pl.ds(off, _L)] = plsc.load_gather(ubrows_v, [jvec, usl])
            pbvals_v[pl.ds(off, _L)] = plsc.load_gather(pbrows_v, [jvec, psl])
    for c in emb_copies:
        c.wait()
    pltpu.sync_copy(urows_v, u_out.at[pl.ds(base, bpw)])
    pltpu.sync_copy(prows_v, p_out.at[pl.ds(base, bpw)])
    pltpu.sync_copy(ubvals_v, ub_out.at[pl.ds(base, bpw)])
    pltpu.sync_copy(pbvals_v, pb_out.at[pl.ds(base, bpw)])


def _tc_mlp_body(u_ref, p_ref, ub_ref, pb_ref, w1u_ref, w1p_ref, b1_ref,
                 w2_ref, b2_ref, w3_ref, b3gb_ref, out_ref):
    u = u_ref[...]
    p = p_ref[...]
    dot = jnp.sum(u * p, axis=1, keepdims=True)
    h = jnp.dot(u, w1u_ref[...], preferred_element_type=jnp.float32)
    h = h + jnp.dot(p, w1p_ref[...], preferred_element_type=jnp.float32)
    h = jnp.maximum(h + b1_ref[...], 0.0)
    h = jnp.maximum(
        jnp.dot(h, w2_ref[...], preferred_element_type=jnp.float32)
        + b2_ref[...], 0.0)
    mlp = jnp.sum(h * w3_ref[...], axis=1, keepdims=True)
    out_ref[...] = dot + mlp + ub_ref[...] + pb_ref[...] + b3gb_ref[...]


def kernel(user_idx, prob_idx, user_emb, prob_emb, user_bias, prob_bias,
           global_bias, W1, b1, W2, b2, W3, b3):
    B = user_idx.shape[0]
    F = user_emb.shape[1]
    H1 = W1.shape[0]
    H2 = W2.shape[0]
    bpw = B // _NW
    k = bpw // _CHUNK

    uidx = user_idx.astype(jnp.int32)
    pidx = prob_idx.astype(jnp.int32)
    uidx3 = uidx.reshape(_NW, k, _CHUNK)
    pidx3 = pidx.reshape(_NW, k, _CHUNK)
    uridx3 = (uidx >> 4).reshape(_NW, k, _CHUNK)
    pridx3 = (pidx >> 4).reshape(_NW, k, _CHUNK)
    ubias16 = user_bias.reshape(-1, _L)
    pbias16 = prob_bias.reshape(-1, _L)

    sc_call = pl.kernel(
        _sc_gather_body,
        out_type=[
            jax.ShapeDtypeStruct((B, F), jnp.float32),
            jax.ShapeDtypeStruct((B, F), jnp.float32),
            jax.ShapeDtypeStruct((B,), jnp.float32),
            jax.ShapeDtypeStruct((B,), jnp.float32),
        ],
        mesh=plsc.VectorSubcoreMesh(core_axis_name="c", subcore_axis_name="s"),
        scratch_types=[
            pltpu.VMEM((k, _CHUNK), jnp.int32),
            pltpu.VMEM((k, _CHUNK), jnp.int32),
            pltpu.VMEM((k, _CHUNK), jnp.int32),
            pltpu.VMEM((k, _CHUNK), jnp.int32),
            pltpu.VMEM((bpw, F), jnp.float32),
            pltpu.VMEM((bpw, F), jnp.float32),
            pltpu.VMEM((bpw, _L), jnp.float32),
            pltpu.VMEM((bpw, _L), jnp.float32),
            pltpu.VMEM((bpw,), jnp.float32),
            pltpu.VMEM((bpw,), jnp.float32),
            pltpu.SemaphoreType.DMA,
            pltpu.SemaphoreType.DMA,
        ],
        compiler_params=pltpu.CompilerParams(
            use_tc_tiling_on_sc=False, needs_layout_passes=False),
    )
    u, p, ub, pb = sc_call(uidx3, pidx3, uridx3, pridx3,
                           user_emb, prob_emb, ubias16, pbias16)

    w1u = W1[:, :F].T
    w1p = W1[:, F:].T
    w2t = W2.T
    b1r = b1.reshape(1, H1)
    b2r = b2.reshape(1, H2)
    b3gb = (b3 + global_bias).reshape(1, 1)

    blk = 2048
    out = pl.pallas_call(
        _tc_mlp_body,
        grid=(B // blk,),
        in_specs=[
            pl.BlockSpec((blk, F), lambda i: (i, 0)),
            pl.BlockSpec((blk, F), lambda i: (i, 0)),
            pl.BlockSpec((blk, 1), lambda i: (i, 0)),
            pl.BlockSpec((blk, 1), lambda i: (i, 0)),
            pl.BlockSpec((F, H1), lambda i: (0, 0)),
            pl.BlockSpec((F, H1), lambda i: (0, 0)),
            pl.BlockSpec((1, H1), lambda i: (0, 0)),
            pl.BlockSpec((H1, H2), lambda i: (0, 0)),
            pl.BlockSpec((1, H2), lambda i: (0, 0)),
            pl.BlockSpec((1, H2), lambda i: (0, 0)),
            pl.BlockSpec((1, 1), lambda i: (0, 0)),
        ],
        out_specs=pl.BlockSpec((blk, 1), lambda i: (i, 0)),
        out_shape=jax.ShapeDtypeStruct((B, 1), jnp.float32),
    )(u, p, ub.reshape(B, 1), pb.reshape(B, 1),
      w1u, w1p, b1r, w2t, b2r, W3, b3gb)
    return out[:, 0]


# speedup vs baseline: 1.1719x; 1.1719x over previous
import jax
import jax.numpy as jnp
from jax import lax
from jax.experimental import pallas as pl
from jax.experimental.pallas import tpu as pltpu
from jax.experimental.pallas import tpu_sc as plsc

_NC = 2
_NS = 16
_NW = _NC * _NS
_L = 16
_SUB = 8
_C = 32


def _sc_emb_gather_body(uidx_hbm, pidx_hbm, uemb_hbm, pemb_hbm,
                        u_out, p_out,
                        uidx_v, pidx_v,
                        tilebuf, urows_v, prows_v, sem):
    f = uemb_hbm.shape[1]
    bpw = 2 * urows_v.shape[0]
    nch = bpw // _C
    wid = lax.axis_index("s") * _NC + lax.axis_index("c")
    base = wid * bpw
    pltpu.sync_copy(uidx_hbm.at[wid], uidx_v)
    pltpu.sync_copy(pidx_hbm.at[wid], pidx_v)
    iota = lax.iota(jnp.int32, _L)

    for idx_v, emb, rows_v in (
        (uidx_v, uemb_hbm, urows_v),
        (pidx_v, pemb_hbm, prows_v),
    ):
        def chunk_body(ch, carry):
            off = ch * _C
            for g in range(_C // _L):
                vidx = idx_v[pl.ds(off + g * _L, _L)]
                for w in range(_L):
                    r = vidx[w]
                    rt = pl.multiple_of((r >> 3) * _SUB, _SUB)
                    pltpu.async_copy(emb.at[pl.ds(rt, _SUB)],
                                     tilebuf.at[g * _L + w], sem)
            for w in range(_C):
                pltpu.make_async_copy(
                    emb.at[pl.ds(0, _SUB)], tilebuf.at[w], sem).wait()
            for g in range(_C // _L):
                ivec = iota + (off + g * _L)
                rowv = iota + g * _L
                svec = idx_v[pl.ds(off + g * _L, _L)] & (_SUB - 1)
                for c in range(f):
                    cvec = jnp.full((_L,), c, jnp.int32)
                    vals = plsc.load_gather(tilebuf, [rowv, svec, cvec])
                    plsc.store_scatter(
                        rows_v, [ivec >> 1, ((ivec & 1) << 6) + c], vals)
            return carry

        lax.fori_loop(0, nch, chunk_body, 0)

    half = bpw // 2
    hbase = pl.multiple_of(base // 2, half)
    pltpu.sync_copy(urows_v, u_out.at[pl.ds(hbase, half)])
    pltpu.sync_copy(prows_v, p_out.at[pl.ds(hbase, half)])


def _sc_bias_gather_body(uidx_hbm, pidx_hbm, uridx_hbm, pridx_hbm,
                         ubias16_hbm, pbias16_hbm, ub_out, pb_out,
                         uidx_v, pidx_v, uridx_v, pridx_v,
                         ubrows_v, pbrows_v, ubvals_v, pbvals_v, sem):
    k = uidx_v.shape[0]
    chunk = uidx_v.shape[1]
    bpw = k * chunk
    wid = lax.axis_index("s") * _NC + lax.axis_index("c")
    base = wid * bpw
    pltpu.sync_copy(uidx_hbm.at[wid], uidx_v)
    pltpu.sync_copy(pidx_hbm.at[wid], pidx_v)
    pltpu.sync_copy(uridx_hbm.at[wid], uridx_v)
    pltpu.sync_copy(pridx_hbm.at[wid], pridx_v)
    copies = []
    for j in range(k):
        sl = pl.ds(j * chunk, chunk)
        copies.append(pltpu.async_copy(
            ubias16_hbm.at[uridx_v.at[j]], ubrows_v.at[sl], sem))
        copies.append(pltpu.async_copy(
            pbias16_hbm.at[pridx_v.at[j]], pbrows_v.at[sl], sem))
    for c in copies:
        c.wait()
    lane_iota = lax.iota(jnp.int32, _L)
    for j in range(k):
        for c in range(chunk // _L):
            off = j * chunk + c * _L
            jvec = off + lane_iota
            usl = uidx_v.at[j][pl.ds(c * _L, _L)] & (_L - 1)
            psl = pidx_v.at[j][pl.ds(c * _L, _L)] & (_L - 1)
            ubvals_v[pl.ds(off, _L)] = plsc.load_gather(ubrows_v, [jvec, usl])
            pbvals_v[pl.ds(off, _L)] = plsc.load_gather(pbrows_v, [jvec, psl])
    pltpu.sync_copy(ubvals_v, ub_out.at[pl.ds(base, bpw)])
    pltpu.sync_copy(pbvals_v, pb_out.at[pl.ds(base, bpw)])


def _tc_mlp_body(u2_ref, p2_ref, ub2_ref, pb2_ref, w1u_ref, w1p_ref, b1_ref,
                 w2_ref, b2_ref, w3_ref, b3gb_ref, out_ref):
    f = w1u_ref.shape[0]
    u2 = u2_ref[...]
    p2 = p2_ref[...]

    def score(u, p):
        dot = jnp.sum(u * p, axis=1, keepdims=True)
        h = jnp.dot(u, w1u_ref[...], preferred_element_type=jnp.float32)
        h = h + jnp.dot(p, w1p_ref[...], preferred_element_type=jnp.float32)
        h = jnp.maximum(h + b1_ref[...], 0.0)
        h = jnp.maximum(
            jnp.dot(h, w2_ref[...], preferred_element_type=jnp.float32)
            + b2_ref[...], 0.0)
        mlp = jnp.sum(h * w3_ref[...], axis=1, keepdims=True)
        return dot + mlp

    s_even = score(u2[:, :f], p2[:, :f])
    s_odd = score(u2[:, f:], p2[:, f:])
    out_ref[...] = (jnp.concatenate([s_even, s_odd], axis=1)
                    + ub2_ref[...] + pb2_ref[...] + b3gb_ref[...])


def kernel(user_idx, prob_idx, user_emb, prob_emb, user_bias, prob_bias,
           global_bias, W1, b1, W2, b2, W3, b3):
    B = user_idx.shape[0]
    F = user_emb.shape[1]
    H1 = W1.shape[0]
    H2 = W2.shape[0]
    bpw = B // _NW
    k = bpw // 128

    uidx = user_idx.astype(jnp.int32)
    pidx = prob_idx.astype(jnp.int32)
    uidx2 = uidx.reshape(_NW, bpw)
    pidx2 = pidx.reshape(_NW, bpw)

    emb_call = pl.kernel(
        _sc_emb_gather_body,
        out_type=[
            jax.ShapeDtypeStruct((B // 2, 2 * F), jnp.float32),
            jax.ShapeDtypeStruct((B // 2, 2 * F), jnp.float32),
        ],
        mesh=plsc.VectorSubcoreMesh(core_axis_name="c", subcore_axis_name="s"),
        scratch_types=[
            pltpu.VMEM((bpw,), jnp.int32),
            pltpu.VMEM((bpw,), jnp.int32),
            pltpu.VMEM((_C, _SUB, F), jnp.float32),
            pltpu.VMEM((bpw // 2, 2 * F), jnp.float32),
            pltpu.VMEM((bpw // 2, 2 * F), jnp.float32),
            pltpu.SemaphoreType.DMA,
        ],
        compiler_params=pltpu.CompilerParams(needs_layout_passes=False),
    )
    u2, p2 = emb_call(uidx2, pidx2, user_emb, prob_emb)

    uidx3 = uidx.reshape(_NW, k, 128)
    pidx3 = pidx.reshape(_NW, k, 128)
    uridx3 = (uidx >> 4).reshape(_NW, k, 128)
    pridx3 = (pidx >> 4).reshape(_NW, k, 128)
    ubias16 = user_bias.reshape(-1, _L)
    pbias16 = prob_bias.reshape(-1, _L)

    bias_call = pl.kernel(
        _sc_bias_gather_body,
        out_type=[
            jax.ShapeDtypeStruct((B,), jnp.float32),
            jax.ShapeDtypeStruct((B,), jnp.float32),
        ],
        mesh=plsc.VectorSubcoreMesh(core_axis_name="c", subcore_axis_name="s"),
        scratch_types=[
            pltpu.VMEM((k, 128), jnp.int32),
            pltpu.VMEM((k, 128), jnp.int32),
            pltpu.VMEM((k, 128), jnp.int32),
            pltpu.VMEM((k, 128), jnp.int32),
            pltpu.VMEM((bpw, _L), jnp.float32),
            pltpu.VMEM((bpw, _L), jnp.float32),
            pltpu.VMEM((bpw,), jnp.float32),
            pltpu.VMEM((bpw,), jnp.float32),
            pltpu.SemaphoreType.DMA,
        ],
        compiler_params=pltpu.CompilerParams(
            use_tc_tiling_on_sc=False, needs_layout_passes=False),
    )
    ub, pb = bias_call(uidx3, pidx3, uridx3, pridx3, ubias16, pbias16)

    w1u = W1[:, :F].T
    w1p = W1[:, F:].T
    w2t = W2.T
    b1r = b1.reshape(1, H1)
    b2r = b2.reshape(1, H2)
    b3gb = (b3 + global_bias).reshape(1, 1)

    blk2 = 1024
    out = pl.pallas_call(
        _tc_mlp_body,
        grid=(B // 2 // blk2,),
        in_specs=[
            pl.BlockSpec((blk2, 2 * F), lambda i: (i, 0)),
            pl.BlockSpec((blk2, 2 * F), lambda i: (i, 0)),
            pl.BlockSpec((blk2, 2), lambda i: (i, 0)),
            pl.BlockSpec((blk2, 2), lambda i: (i, 0)),
            pl.BlockSpec((F, H1), lambda i: (0, 0)),
            pl.BlockSpec((F, H1), lambda i: (0, 0)),
            pl.BlockSpec((1, H1), lambda i: (0, 0)),
            pl.BlockSpec((H1, H2), lambda i: (0, 0)),
            pl.BlockSpec((1, H2), lambda i: (0, 0)),
            pl.BlockSpec((1, H2), lambda i: (0, 0)),
            pl.BlockSpec((1, 1), lambda i: (0, 0)),
        ],
        out_specs=pl.BlockSpec((blk2, 2), lambda i: (i, 0)),
        out_shape=jax.ShapeDtypeStruct((B // 2, 2), jnp.float32),
    )(u2, p2, ub.reshape(B // 2, 2), pb.reshape(B // 2, 2),
      w1u, w1p, b1r, w2t, b2r, W3, b3gb)
    return out.reshape(B)
